# baseline (speedup 1.0000x reference)
import functools

import jax
import jax.numpy as jnp
from jax import lax
from jax.experimental import pallas as pl
from jax.experimental.pallas import tpu as pltpu
from jax.experimental.pallas import tpu_sc as plsc


def _sc_gather(x, idx):
    B = idx.shape[0]
    D = x.shape[1]
    info = plsc.get_sparse_core_info()
    NC, NS = info.num_cores, info.num_subcores
    NW = NC * NS
    b_per_w = B // NW
    mesh = plsc.VectorSubcoreMesh(core_axis_name="c", subcore_axis_name="s")

    @functools.partial(
        pl.kernel,
        mesh=mesh,
        out_type=jax.ShapeDtypeStruct((B, D), x.dtype),
        scratch_types=[
            pltpu.VMEM((b_per_w,), jnp.int32),
            pltpu.VMEM((b_per_w, D), x.dtype),
            pltpu.SemaphoreType.DMA,
        ],
    )
    def k(x_hbm, idx_hbm, out_hbm, idx_v, rows_v, sem):
        wid = lax.axis_index("s") * NC + lax.axis_index("c")
        base = wid * b_per_w
        pltpu.sync_copy(idx_hbm.at[pl.ds(base, b_per_w)], idx_v)
        pltpu.async_copy(x_hbm.at[idx_v], rows_v, sem).wait()
        pltpu.sync_copy(rows_v, out_hbm.at[pl.ds(base, b_per_w)])

    return k(x, idx)


def _tc_gather(x, idx):
    B = idx.shape[0]
    D = x.shape[1]

    def body(idx_ref, x_ref, out_ref, sem):
        def issue(i, c):
            pltpu.make_async_copy(x_ref.at[idx_ref[i]], out_ref.at[i], sem).start()
            return c

        lax.fori_loop(0, B, issue, 0)
        pltpu.make_async_copy(x_ref.at[pl.ds(0, B)], out_ref, sem).wait()

    return pl.pallas_call(
        body,
        in_specs=[
            pl.BlockSpec(memory_space=pltpu.MemorySpace.SMEM),
            pl.BlockSpec(memory_space=pl.ANY),
        ],
        out_specs=pl.BlockSpec(memory_space=pl.ANY),
        out_shape=jax.ShapeDtypeStruct((B, D), x.dtype),
        scratch_shapes=[pltpu.SemaphoreType.DMA],
    )(idx, x)


_R_SC = 512


def kernel(x, component_starts):
    idx = component_starts.astype(jnp.int32)
    out_sc = _sc_gather(x, idx[:_R_SC])
    out_tc = _tc_gather(x, idx[_R_SC:])
    return jnp.concatenate([out_sc, out_tc], axis=0)


# speedup vs baseline: 1.3598x; 1.3598x over previous
import functools

import jax
import jax.numpy as jnp
from jax import lax
from jax.experimental import pallas as pl
from jax.experimental.pallas import tpu as pltpu
from jax.experimental.pallas import tpu_sc as plsc


def _gather_rows(x, idx):
    B = idx.shape[0]
    D = x.shape[1]
    info = plsc.get_sparse_core_info()
    NC, NS = info.num_cores, info.num_subcores
    NW = NC * NS
    b_per_w = B // NW
    h = b_per_w // 2
    mesh = plsc.VectorSubcoreMesh(core_axis_name="c", subcore_axis_name="s")

    @functools.partial(
        pl.kernel,
        mesh=mesh,
        out_type=jax.ShapeDtypeStruct((B, D), x.dtype),
        scratch_types=[
            pltpu.VMEM((b_per_w,), jnp.int32),
            pltpu.VMEM((b_per_w, D), x.dtype),
            pltpu.SemaphoreType.DMA,
            pltpu.SemaphoreType.DMA,
            pltpu.SemaphoreType.DMA,
            pltpu.SemaphoreType.DMA,
        ],
    )
    def k(x_hbm, idx_hbm, out_hbm, idx_v, rows_v, isem0, isem1, gsem0, gsem1):
        wid = lax.axis_index("s") * NC + lax.axis_index("c")
        base = wid * b_per_w
        i0 = pltpu.async_copy(
            idx_hbm.at[pl.ds(base, h)], idx_v.at[pl.ds(0, h)], isem0
        )
        i1 = pltpu.async_copy(
            idx_hbm.at[pl.ds(base + h, h)], idx_v.at[pl.ds(h, h)], isem1
        )
        i0.wait()
        g0 = pltpu.async_copy(
            x_hbm.at[idx_v.at[pl.ds(0, h)]], rows_v.at[pl.ds(0, h)], gsem0
        )
        i1.wait()
        g1 = pltpu.async_copy(
            x_hbm.at[idx_v.at[pl.ds(h, h)]], rows_v.at[pl.ds(h, h)], gsem1
        )
        g0.wait()
        w0 = pltpu.async_copy(
            rows_v.at[pl.ds(0, h)], out_hbm.at[pl.ds(base, h)], isem0
        )
        g1.wait()
        w1 = pltpu.async_copy(
            rows_v.at[pl.ds(h, h)], out_hbm.at[pl.ds(base + h, h)], isem1
        )
        w0.wait()
        w1.wait()

    return k(x, idx)


def kernel(x, component_starts):
    idx = component_starts.astype(jnp.int32)
    return _gather_rows(x, idx)


# speedup vs baseline: 1.5821x; 1.1634x over previous
import functools

import jax
import jax.numpy as jnp
from jax import lax
from jax.experimental import pallas as pl
from jax.experimental.pallas import tpu as pltpu
from jax.experimental.pallas import tpu_sc as plsc


def _gather_rows(x, idx):
    B = idx.shape[0]
    D = x.shape[1]
    info = plsc.get_sparse_core_info()
    NC, NS = info.num_cores, info.num_subcores
    NW = NC * NS
    b_per_w = B // NW
    h = b_per_w // 2
    mesh = plsc.VectorSubcoreMesh(core_axis_name="c", subcore_axis_name="s", num_cores=1)

    @functools.partial(
        pl.kernel,
        mesh=mesh,
        out_type=jax.ShapeDtypeStruct((B, D), x.dtype),
        scratch_types=[
            pltpu.VMEM((b_per_w,), jnp.int32),
            pltpu.VMEM((b_per_w, D), x.dtype),
            pltpu.SemaphoreType.DMA,
            pltpu.SemaphoreType.DMA,
            pltpu.SemaphoreType.DMA,
            pltpu.SemaphoreType.DMA,
        ],
    )
    def k(x_hbm, idx_hbm, out_hbm, idx_v, rows_v, isem0, isem1, gsem0, gsem1):
        wid = lax.axis_index("s") * NC + lax.axis_index("c")
        base = wid * b_per_w
        i0 = pltpu.async_copy(
            idx_hbm.at[pl.ds(base, h)], idx_v.at[pl.ds(0, h)], isem0
        )
        i1 = pltpu.async_copy(
            idx_hbm.at[pl.ds(base + h, h)], idx_v.at[pl.ds(h, h)], isem1
        )
        i0.wait()
        g0 = pltpu.async_copy(
            x_hbm.at[idx_v.at[pl.ds(0, h)]], rows_v.at[pl.ds(0, h)], gsem0
        )
        i1.wait()
        g1 = pltpu.async_copy(
            x_hbm.at[idx_v.at[pl.ds(h, h)]], rows_v.at[pl.ds(h, h)], gsem1
        )
        g0.wait()
        w0 = pltpu.async_copy(
            rows_v.at[pl.ds(0, h)], out_hbm.at[pl.ds(base, h)], isem0
        )
        g1.wait()
        w1 = pltpu.async_copy(
            rows_v.at[pl.ds(h, h)], out_hbm.at[pl.ds(base + h, h)], isem1
        )
        w0.wait()
        w1.wait()

    return k(x, idx)


def kernel(x, component_starts):
    idx = component_starts.astype(jnp.int32)
    return _gather_rows(x, idx)
